# baseline (speedup 1.0000x reference)
import functools
import math

import jax
import jax.numpy as jnp
from jax import lax
from jax.experimental import pallas as pl
from jax.experimental.pallas import tpu as pltpu
from jax.experimental.pallas import tpu_sc as plsc

D_MODEL = 64
SCALE = math.sqrt(D_MODEL)
NUM_CORES = 2
NUM_SUBCORES = 16
NUM_WORKERS = NUM_CORES * NUM_SUBCORES
LANES = 16


@jax.jit
def _embed(x, emb):
  b, l = x.shape
  rows_per_w = b // NUM_WORKERS
  n_pairs = rows_per_w // 2

  mesh = plsc.VectorSubcoreMesh(
      core_axis_name="c", subcore_axis_name="s",
      num_cores=NUM_CORES, num_subcores=NUM_SUBCORES)

  @functools.partial(
      pl.kernel,
      mesh=mesh,
      out_type=jax.ShapeDtypeStruct((b, l, D_MODEL), jnp.float32),
      compiler_params=pltpu.CompilerParams(use_tc_tiling_on_sc=False),
      scratch_types=[
          pltpu.VMEM((rows_per_w, 200), jnp.int32),
          pltpu.VMEM((200, D_MODEL), jnp.float32),
          pltpu.VMEM((200, D_MODEL), jnp.float32),
          pltpu.SemaphoreType.DMA,
          pltpu.SemaphoreType.DMA,
          pltpu.SemaphoreType.DMA,
          pltpu.SemaphoreType.DMA,
      ],
  )
  def k(x_hbm, emb_hbm, out_hbm, idx_v, rows0, rows1, g0, g1, s0, s1):
    wid = lax.axis_index("s") * NUM_CORES + lax.axis_index("c")
    base = wid * rows_per_w
    rows = (rows0, rows1)
    gsem = (g0, g1)
    ssem = (s0, s1)

    pltpu.sync_copy(x_hbm.at[pl.ds(base, rows_per_w)], idx_v)

    def gather(r, bb):
      return pltpu.make_async_copy(
          emb_hbm.at[idx_v.at[r]], rows[bb], gsem[bb])

    def store(r, bb):
      return pltpu.make_async_copy(rows[bb], out_hbm.at[base + r], ssem[bb])

    gather(0, 0).start()

    def pair_body(i, carry):
      for bb in range(2):
        r = 2 * i + bb
        other = 1 - bb

        if bb == 0:
          @pl.when(i > 0)
          def _():
            store(r - 1, other).wait()
          gather(r + 1, other).start()
        else:
          store(r - 1, other).wait()
          @pl.when(i < n_pairs - 1)
          def _():
            gather(r + 1, other).start()

        gather(r, bb).wait()

        @plsc.parallel_loop(0, 200, step=1, unroll=4)
        def _(rr):
          for c in range(D_MODEL // LANES):
            sl = pl.ds(c * LANES, LANES)
            rows[bb][rr, sl] = rows[bb][rr, sl] * SCALE

        store(r, bb).start()
      return carry

    lax.fori_loop(0, n_pairs, pair_body, 0)
    store(rows_per_w - 1, 1).wait()

  return k(x, emb)


def kernel(x, emb):
  return _embed(x.astype(jnp.int32), emb)
